# baseline (device time: 784549 ns/iter reference)
import jax
import jax.numpy as jnp
from jax import lax
from jax.experimental import pallas as pl
from jax.experimental.pallas import tpu as pltpu

N_DEV = 32
E_LOCAL = 4
N_TOK = 2048
D = 512
H = 1024
HH = H // 2
ROWS = N_TOK // N_DEV
CAP = 128


def kernel(x, router_W, route_idx, expert_W, shared_W):
    k = lax.axis_index("i")

    scores = jnp.dot(x, router_W, preferred_element_type=jnp.float32)
    probs = jax.nn.softmax(scores, axis=-1)
    oh128 = (jnp.arange(probs.shape[1])[None, :] == route_idx)
    p = jnp.sum(probs * oh128.astype(jnp.float32), axis=1)

    local_j = route_idx[:, 0] - E_LOCAL * k
    oh4 = local_j[:, None] == jnp.arange(E_LOCAL)[None, :]
    ranks = jnp.cumsum(oh4.astype(jnp.int32), axis=0) - 1
    rank = jnp.sum(ranks * oh4, axis=1)
    is_local = jnp.any(oh4, axis=1)
    jj = jnp.clip(local_j, 0, E_LOCAL - 1)
    slot = jnp.where(is_local & (rank < CAP), jj * CAP + rank, E_LOCAL * CAP)

    xp = x * p[:, None]
    xc = jnp.zeros((E_LOCAL * CAP, D), jnp.float32).at[slot].add(xp, mode="drop")
    yc = jax.lax.dot_general(
        xc.reshape(E_LOCAL, CAP, D), expert_W,
        dimension_numbers=(((2,), (1,)), ((0,), (0,))),
        preferred_element_type=jnp.float32,
    )
    tok = jnp.zeros((E_LOCAL * CAP,), jnp.int32).at[slot].add(
        jnp.arange(N_TOK, dtype=jnp.int32), mode="drop"
    )
    partial = jnp.zeros((N_TOK, H), jnp.float32).at[tok].add(
        yc.reshape(E_LOCAL * CAP, H), mode="drop"
    )

    shared_own = jnp.dot(
        lax.dynamic_slice(x, (ROWS * k, 0), (ROWS, D)),
        shared_W,
        preferred_element_type=jnp.float32,
    )

    def body(partial_ref, sh_ref, out_ref,
             acc_r, acc_l, recv_r, recv_l,
             send_r_sems, recv_r_sems, send_l_sems, recv_l_sems):
        my = lax.axis_index("i")
        left = (my - 1 + N_DEV) % N_DEV
        right = (my + 1) % N_DEV

        barrier_sem = pltpu.get_barrier_semaphore()
        for nbr in (left, right):
            pl.semaphore_signal(
                barrier_sem, inc=1,
                device_id=(nbr,), device_id_type=pl.DeviceIdType.MESH,
            )
        pl.semaphore_wait(barrier_sem, 2)

        c0r = (my - 1 + N_DEV) % N_DEV
        c0l = (my + 1) % N_DEV
        acc_r[0, :, :] = partial_ref[pl.ds(c0r * ROWS, ROWS), 0:HH]
        acc_l[0, :, :] = partial_ref[pl.ds(c0l * ROWS, ROWS), HH:H]
        for s in range(N_DEV - 1):
            slot_ = s % 2
            rdma_r = pltpu.make_async_remote_copy(
                src_ref=acc_r.at[slot_],
                dst_ref=recv_r.at[s],
                send_sem=send_r_sems.at[s],
                recv_sem=recv_r_sems.at[s],
                device_id=(right,),
                device_id_type=pl.DeviceIdType.MESH,
            )
            rdma_l = pltpu.make_async_remote_copy(
                src_ref=acc_l.at[slot_],
                dst_ref=recv_l.at[s],
                send_sem=send_l_sems.at[s],
                recv_sem=recv_l_sems.at[s],
                device_id=(left,),
                device_id_type=pl.DeviceIdType.MESH,
            )
            rdma_r.start()
            rdma_l.start()
            rdma_r.wait()
            rdma_l.wait()
            if s < N_DEV - 2:
                cr = (my - s - 2 + N_DEV) % N_DEV
                cl = (my + s + 2) % N_DEV
                acc_r[1 - slot_, :, :] = (
                    recv_r[s] + partial_ref[pl.ds(cr * ROWS, ROWS), 0:HH]
                )
                acc_l[1 - slot_, :, :] = (
                    recv_l[s] + partial_ref[pl.ds(cl * ROWS, ROWS), HH:H]
                )
            else:
                own = partial_ref[pl.ds(my * ROWS, ROWS), :] + sh_ref[...]
                out_ref[:, 0:HH] = recv_r[s] + own[:, 0:HH]
                out_ref[:, HH:H] = recv_l[s] + own[:, HH:H]

    return pl.pallas_call(
        body,
        out_shape=jax.ShapeDtypeStruct((ROWS, H), jnp.float32),
        in_specs=[
            pl.BlockSpec(memory_space=pltpu.VMEM),
            pl.BlockSpec(memory_space=pltpu.VMEM),
        ],
        out_specs=pl.BlockSpec(memory_space=pltpu.VMEM),
        scratch_shapes=[
            pltpu.VMEM((2, ROWS, HH), jnp.float32),
            pltpu.VMEM((2, ROWS, HH), jnp.float32),
            pltpu.VMEM((N_DEV - 1, ROWS, HH), jnp.float32),
            pltpu.VMEM((N_DEV - 1, ROWS, HH), jnp.float32),
            pltpu.SemaphoreType.DMA((N_DEV - 1,)),
            pltpu.SemaphoreType.DMA((N_DEV - 1,)),
            pltpu.SemaphoreType.DMA((N_DEV - 1,)),
            pltpu.SemaphoreType.DMA((N_DEV - 1,)),
        ],
        compiler_params=pltpu.CompilerParams(collective_id=0),
    )(partial, shared_own)


# device time: 186636 ns/iter; 4.2036x vs baseline; 4.2036x over previous
import jax
import jax.numpy as jnp
from jax import lax
from jax.experimental import pallas as pl
from jax.experimental.pallas import tpu as pltpu

N_DEV = 32
E_LOCAL = 4
N_TOK = 2048
D = 512
H = 1024
HH = H // 2
ROWS = N_TOK // N_DEV


def kernel(x, router_W, route_idx, expert_W, shared_W):
    k = lax.axis_index("i")

    scores = jnp.dot(x, router_W, preferred_element_type=jnp.float32)
    probs = jax.nn.softmax(scores, axis=-1)
    oh128 = (jnp.arange(probs.shape[1])[None, :] == route_idx)
    p = jnp.sum(probs * oh128.astype(jnp.float32), axis=1)

    local_j = route_idx[:, 0] - E_LOCAL * k
    oh = (local_j[:, None] == jnp.arange(E_LOCAL)[None, :])
    gate = oh.astype(jnp.float32) * p[:, None]

    shared_own = jnp.dot(
        lax.dynamic_slice(x, (ROWS * k, 0), (ROWS, D)),
        shared_W,
        preferred_element_type=jnp.float32,
    )

    def body(x_ref, gate_ref, w_ref, sh_ref, out_ref,
             acc_r, acc_l, recv_r, recv_l,
             send_r_sems, recv_r_sems, send_l_sems, recv_l_sems):
        my = lax.axis_index("i")
        left = (my - 1 + N_DEV) % N_DEV
        right = (my + 1) % N_DEV

        barrier_sem = pltpu.get_barrier_semaphore()
        for nbr in (left, right):
            pl.semaphore_signal(
                barrier_sem, inc=1,
                device_id=(nbr,), device_id_type=pl.DeviceIdType.MESH,
            )
        pl.semaphore_wait(barrier_sem, 2)

        def chunk_partial(c, lo, hi):
            xrow = x_ref[pl.ds(c * ROWS, ROWS), :]
            g = gate_ref[pl.ds(c * ROWS, ROWS), :]
            acc = jnp.zeros((ROWS, hi - lo), jnp.float32)
            for j in range(E_LOCAL):
                acc = acc + jnp.dot(
                    xrow * g[:, j:j + 1], w_ref[j, :, lo:hi],
                    preferred_element_type=jnp.float32,
                )
            return acc

        c0r = (my - 1 + N_DEV) % N_DEV
        c0l = (my + 1) % N_DEV
        acc_r[0, :, :] = chunk_partial(c0r, 0, HH)
        acc_l[0, :, :] = chunk_partial(c0l, HH, H)
        for s in range(N_DEV - 1):
            slot = s % 2
            rdma_r = pltpu.make_async_remote_copy(
                src_ref=acc_r.at[slot],
                dst_ref=recv_r.at[s],
                send_sem=send_r_sems.at[s],
                recv_sem=recv_r_sems.at[s],
                device_id=(right,),
                device_id_type=pl.DeviceIdType.MESH,
            )
            rdma_l = pltpu.make_async_remote_copy(
                src_ref=acc_l.at[slot],
                dst_ref=recv_l.at[s],
                send_sem=send_l_sems.at[s],
                recv_sem=recv_l_sems.at[s],
                device_id=(left,),
                device_id_type=pl.DeviceIdType.MESH,
            )
            rdma_r.start()
            rdma_l.start()
            if s < N_DEV - 2:
                cr = (my - s - 2 + N_DEV) % N_DEV
                cl = (my + s + 2) % N_DEV
                part_r = chunk_partial(cr, 0, HH)
                part_l = chunk_partial(cl, HH, H)
                rdma_r.wait()
                rdma_l.wait()
                acc_r[1 - slot, :, :] = recv_r[s] + part_r
                acc_l[1 - slot, :, :] = recv_l[s] + part_l
            else:
                own = chunk_partial(my, 0, H) + sh_ref[...]
                rdma_r.wait()
                rdma_l.wait()
                out_ref[:, 0:HH] = recv_r[s] + own[:, 0:HH]
                out_ref[:, HH:H] = recv_l[s] + own[:, HH:H]

    return pl.pallas_call(
        body,
        out_shape=jax.ShapeDtypeStruct((ROWS, H), jnp.float32),
        in_specs=[
            pl.BlockSpec(memory_space=pltpu.VMEM),
            pl.BlockSpec(memory_space=pltpu.VMEM),
            pl.BlockSpec(memory_space=pltpu.VMEM),
            pl.BlockSpec(memory_space=pltpu.VMEM),
        ],
        out_specs=pl.BlockSpec(memory_space=pltpu.VMEM),
        scratch_shapes=[
            pltpu.VMEM((2, ROWS, HH), jnp.float32),
            pltpu.VMEM((2, ROWS, HH), jnp.float32),
            pltpu.VMEM((N_DEV - 1, ROWS, HH), jnp.float32),
            pltpu.VMEM((N_DEV - 1, ROWS, HH), jnp.float32),
            pltpu.SemaphoreType.DMA((N_DEV - 1,)),
            pltpu.SemaphoreType.DMA((N_DEV - 1,)),
            pltpu.SemaphoreType.DMA((N_DEV - 1,)),
            pltpu.SemaphoreType.DMA((N_DEV - 1,)),
        ],
        compiler_params=pltpu.CompilerParams(collective_id=0),
    )(x, gate, expert_W, shared_own)


# device time: 83329 ns/iter; 9.4151x vs baseline; 2.2397x over previous
import jax
import jax.numpy as jnp
from jax import lax
from jax.experimental import pallas as pl
from jax.experimental.pallas import tpu as pltpu

N_DEV = 32
E_LOCAL = 4
N_TOK = 2048
D = 512
H = 1024
HH = H // 2
ROWS = N_TOK // N_DEV
NZ = 4
NQ = 8
SROWS = NZ * ROWS

CYC2Q = (0, 3, 4, 7, 6, 5, 2, 1)
Q2CYC = (0, 7, 6, 1, 2, 5, 4, 3)


def _lut(table, idx):
    r = 0
    for t, v in enumerate(table):
        r = r + v * (idx == t)
    return r


def kernel(x, router_W, route_idx, expert_W, shared_W):
    k = lax.axis_index("i")

    scores = jnp.dot(x, router_W, preferred_element_type=jnp.float32)
    probs = jax.nn.softmax(scores, axis=-1)
    oh128 = (jnp.arange(probs.shape[1])[None, :] == route_idx)
    p = jnp.sum(probs * oh128.astype(jnp.float32), axis=1)

    local_j = route_idx[:, 0] - E_LOCAL * k
    oh = (local_j[:, None] == jnp.arange(E_LOCAL)[None, :])
    gate = oh.astype(jnp.float32) * p[:, None]

    x4 = x.reshape(NZ, NQ * ROWS, D)
    gate4 = gate.reshape(NZ, NQ * ROWS, E_LOCAL)

    shared_own = jnp.dot(
        lax.dynamic_slice(x, (ROWS * k, 0), (ROWS, D)),
        shared_W,
        preferred_element_type=jnp.float32,
    )

    def body(x4_ref, gate4_ref, w_ref, sh_ref, out_ref,
             acc_r, acc_l, recv_r, recv_l, s_ref, acc2, recv2,
             s1r_send, s1r_recv, s1l_send, s1l_recv, s2_send, s2_recv):
        my = lax.axis_index("i")
        z = my // NQ
        q = my % NQ
        cm = _lut(Q2CYC, q)
        cyc_succ = z * NQ + _lut(CYC2Q, (cm + 1) % NQ)
        cyc_pred = z * NQ + _lut(CYC2Q, (cm - 1 + NQ) % NQ)
        zsucc = ((z + 1) % NZ) * NQ + q
        zpred = ((z - 1 + NZ) % NZ) * NQ + q

        barrier_sem = pltpu.get_barrier_semaphore()
        for nbr in (cyc_pred, cyc_succ, zpred, zsucc):
            pl.semaphore_signal(
                barrier_sem, inc=1,
                device_id=(nbr,), device_id_type=pl.DeviceIdType.MESH,
            )
        pl.semaphore_wait(barrier_sem, 4)

        def sc_partial_half(qq, lo):
            xz = x4_ref[:, pl.ds(qq * ROWS, ROWS), :]
            res = jnp.zeros((SROWS, HH), jnp.float32)
            for j in range(E_LOCAL):
                g = gate4_ref[:, pl.ds(qq * ROWS, ROWS), j:j + 1]
                xg = (xz * g).reshape(SROWS, D)
                res = res + jnp.dot(
                    xg, w_ref[j, :, lo:lo + HH],
                    preferred_element_type=jnp.float32,
                )
            return res

        q0r = _lut(CYC2Q, (cm - 1 + NQ) % NQ)
        q0l = _lut(CYC2Q, (cm + 1) % NQ)
        acc_r[0, :, :] = sc_partial_half(q0r, 0)
        acc_l[0, :, :] = sc_partial_half(q0l, HH)
        for s in range(NQ - 1):
            slot = s % 2
            rdma_r = pltpu.make_async_remote_copy(
                src_ref=acc_r.at[slot],
                dst_ref=recv_r.at[s],
                send_sem=s1r_send.at[s],
                recv_sem=s1r_recv.at[s],
                device_id=(cyc_succ,),
                device_id_type=pl.DeviceIdType.MESH,
            )
            rdma_l = pltpu.make_async_remote_copy(
                src_ref=acc_l.at[slot],
                dst_ref=recv_l.at[s],
                send_sem=s1l_send.at[s],
                recv_sem=s1l_recv.at[s],
                device_id=(cyc_pred,),
                device_id_type=pl.DeviceIdType.MESH,
            )
            rdma_r.start()
            rdma_l.start()
            qr = _lut(CYC2Q, (cm - s - 2 + 2 * NQ) % NQ)
            ql = _lut(CYC2Q, (cm + s + 2) % NQ)
            part_r = sc_partial_half(qr, 0)
            part_l = sc_partial_half(ql, HH)
            rdma_r.wait()
            rdma_l.wait()
            if s < NQ - 2:
                acc_r[1 - slot, :, :] = recv_r[s] + part_r
                acc_l[1 - slot, :, :] = recv_l[s] + part_l
            else:
                s_ref[:, 0:HH] = recv_r[s] + part_r
                s_ref[:, HH:H] = recv_l[s] + part_l

        p0 = (z - 1 + NZ) % NZ
        acc2[0, :, :] = s_ref[pl.ds(p0 * ROWS, ROWS), :]
        for s in range(NZ - 1):
            slot = s % 2
            rdma2 = pltpu.make_async_remote_copy(
                src_ref=acc2.at[slot],
                dst_ref=recv2.at[s],
                send_sem=s2_send.at[s],
                recv_sem=s2_recv.at[s],
                device_id=(zsucc,),
                device_id_type=pl.DeviceIdType.MESH,
            )
            rdma2.start()
            rdma2.wait()
            if s < NZ - 2:
                pz = (z - s - 2 + NZ) % NZ
                acc2[1 - slot, :, :] = recv2[s] + s_ref[pl.ds(pz * ROWS, ROWS), :]
            else:
                out_ref[...] = (
                    recv2[s] + s_ref[pl.ds(z * ROWS, ROWS), :] + sh_ref[...]
                )

    return pl.pallas_call(
        body,
        out_shape=jax.ShapeDtypeStruct((ROWS, H), jnp.float32),
        in_specs=[
            pl.BlockSpec(memory_space=pltpu.VMEM),
            pl.BlockSpec(memory_space=pltpu.VMEM),
            pl.BlockSpec(memory_space=pltpu.VMEM),
            pl.BlockSpec(memory_space=pltpu.VMEM),
        ],
        out_specs=pl.BlockSpec(memory_space=pltpu.VMEM),
        scratch_shapes=[
            pltpu.VMEM((2, SROWS, HH), jnp.float32),
            pltpu.VMEM((2, SROWS, HH), jnp.float32),
            pltpu.VMEM((NQ - 1, SROWS, HH), jnp.float32),
            pltpu.VMEM((NQ - 1, SROWS, HH), jnp.float32),
            pltpu.VMEM((SROWS, H), jnp.float32),
            pltpu.VMEM((2, ROWS, H), jnp.float32),
            pltpu.VMEM((NZ - 1, ROWS, H), jnp.float32),
            pltpu.SemaphoreType.DMA((NQ - 1,)),
            pltpu.SemaphoreType.DMA((NQ - 1,)),
            pltpu.SemaphoreType.DMA((NQ - 1,)),
            pltpu.SemaphoreType.DMA((NQ - 1,)),
            pltpu.SemaphoreType.DMA((NZ - 1,)),
            pltpu.SemaphoreType.DMA((NZ - 1,)),
        ],
        compiler_params=pltpu.CompilerParams(collective_id=0),
    )(x4, gate4, expert_W, shared_own)


# device time: 59272 ns/iter; 13.2364x vs baseline; 1.4059x over previous
import jax
import jax.numpy as jnp
from jax import lax
from jax.experimental import pallas as pl
from jax.experimental.pallas import tpu as pltpu

N_DEV = 32
E_LOCAL = 4
N_TOK = 2048
D = 512
H = 1024
HH = H // 2
ROWS = N_TOK // N_DEV
NZ = 4
NQ = 8
SROWS = NZ * ROWS

CYC2Q = (0, 3, 4, 7, 6, 5, 2, 1)
Q2CYC = (0, 7, 6, 1, 2, 5, 4, 3)


def _lut(table, idx):
    r = 0
    for t, v in enumerate(table):
        r = r + v * (idx == t)
    return r


def kernel(x, router_W, route_idx, expert_W, shared_W):
    k = lax.axis_index("i")

    scores = jnp.dot(x, router_W, preferred_element_type=jnp.float32)
    probs = jax.nn.softmax(scores, axis=-1)
    oh128 = (jnp.arange(probs.shape[1])[None, :] == route_idx)
    p = jnp.sum(probs * oh128.astype(jnp.float32), axis=1)

    local_j = route_idx[:, 0] - E_LOCAL * k
    oh = (local_j[:, None] == jnp.arange(E_LOCAL)[None, :])
    gate = oh.astype(jnp.float32) * p[:, None]

    x4 = x.reshape(NZ, NQ * ROWS, D).astype(jnp.bfloat16)
    gate4 = gate.reshape(NZ, NQ * ROWS, E_LOCAL).astype(jnp.bfloat16)
    expert_W = expert_W.astype(jnp.bfloat16)

    shared_own = jnp.dot(
        lax.dynamic_slice(x, (ROWS * k, 0), (ROWS, D)),
        shared_W,
        preferred_element_type=jnp.float32,
    )

    def body(x4_ref, gate4_ref, w_ref, sh_ref, out_ref,
             acc_r, acc_l, recv_r, recv_l, s_ref, acc2, recv2,
             s1r_send, s1r_recv, s1l_send, s1l_recv, s2_send, s2_recv):
        my = lax.axis_index("i")
        z = my // NQ
        q = my % NQ
        cm = _lut(Q2CYC, q)
        cyc_succ = z * NQ + _lut(CYC2Q, (cm + 1) % NQ)
        cyc_pred = z * NQ + _lut(CYC2Q, (cm - 1 + NQ) % NQ)
        zsucc = ((z + 1) % NZ) * NQ + q
        zpred = ((z - 1 + NZ) % NZ) * NQ + q

        barrier_sem = pltpu.get_barrier_semaphore()
        for nbr in (cyc_pred, cyc_succ, zpred, zsucc):
            pl.semaphore_signal(
                barrier_sem, inc=1,
                device_id=(nbr,), device_id_type=pl.DeviceIdType.MESH,
            )
        pl.semaphore_wait(barrier_sem, 4)

        def sc_partial_half(qq, lo):
            xz = x4_ref[:, pl.ds(qq * ROWS, ROWS), :]
            res = jnp.zeros((SROWS, HH), jnp.float32)
            for j in range(E_LOCAL):
                g = gate4_ref[:, pl.ds(qq * ROWS, ROWS), j:j + 1]
                xg = (xz * g).reshape(SROWS, D)
                res = res + jnp.dot(
                    xg, w_ref[j, :, lo:lo + HH],
                    preferred_element_type=jnp.float32,
                )
            return res

        q0r = _lut(CYC2Q, (cm - 1 + NQ) % NQ)
        q0l = _lut(CYC2Q, (cm + 1) % NQ)
        acc_r[0, :, :] = sc_partial_half(q0r, 0).astype(jnp.bfloat16)
        acc_l[0, :, :] = sc_partial_half(q0l, HH).astype(jnp.bfloat16)
        for s in range(NQ - 1):
            slot = s % 2
            rdma_r = pltpu.make_async_remote_copy(
                src_ref=acc_r.at[slot],
                dst_ref=recv_r.at[s],
                send_sem=s1r_send.at[s],
                recv_sem=s1r_recv.at[s],
                device_id=(cyc_succ,),
                device_id_type=pl.DeviceIdType.MESH,
            )
            rdma_l = pltpu.make_async_remote_copy(
                src_ref=acc_l.at[slot],
                dst_ref=recv_l.at[s],
                send_sem=s1l_send.at[s],
                recv_sem=s1l_recv.at[s],
                device_id=(cyc_pred,),
                device_id_type=pl.DeviceIdType.MESH,
            )
            rdma_r.start()
            rdma_l.start()
            qr = _lut(CYC2Q, (cm - s - 2 + 2 * NQ) % NQ)
            ql = _lut(CYC2Q, (cm + s + 2) % NQ)
            part_r = sc_partial_half(qr, 0)
            part_l = sc_partial_half(ql, HH)
            rdma_r.wait()
            rdma_l.wait()
            if s < NQ - 2:
                acc_r[1 - slot, :, :] = (
                    recv_r[s].astype(jnp.float32) + part_r
                ).astype(jnp.bfloat16)
                acc_l[1 - slot, :, :] = (
                    recv_l[s].astype(jnp.float32) + part_l
                ).astype(jnp.bfloat16)
            else:
                s_ref[:, 0:HH] = recv_r[s].astype(jnp.float32) + part_r
                s_ref[:, HH:H] = recv_l[s].astype(jnp.float32) + part_l

        p0 = (z - 1 + NZ) % NZ
        acc2[0, :, :] = s_ref[pl.ds(p0 * ROWS, ROWS), :].astype(jnp.bfloat16)
        for s in range(NZ - 1):
            slot = s % 2
            rdma2 = pltpu.make_async_remote_copy(
                src_ref=acc2.at[slot],
                dst_ref=recv2.at[s],
                send_sem=s2_send.at[s],
                recv_sem=s2_recv.at[s],
                device_id=(zsucc,),
                device_id_type=pl.DeviceIdType.MESH,
            )
            rdma2.start()
            rdma2.wait()
            if s < NZ - 2:
                pz = (z - s - 2 + NZ) % NZ
                acc2[1 - slot, :, :] = (
                    recv2[s].astype(jnp.float32)
                    + s_ref[pl.ds(pz * ROWS, ROWS), :]
                ).astype(jnp.bfloat16)
            else:
                out_ref[...] = (
                    recv2[s].astype(jnp.float32)
                    + s_ref[pl.ds(z * ROWS, ROWS), :]
                    + sh_ref[...]
                )

    return pl.pallas_call(
        body,
        out_shape=jax.ShapeDtypeStruct((ROWS, H), jnp.float32),
        in_specs=[
            pl.BlockSpec(memory_space=pltpu.VMEM),
            pl.BlockSpec(memory_space=pltpu.VMEM),
            pl.BlockSpec(memory_space=pltpu.VMEM),
            pl.BlockSpec(memory_space=pltpu.VMEM),
        ],
        out_specs=pl.BlockSpec(memory_space=pltpu.VMEM),
        scratch_shapes=[
            pltpu.VMEM((2, SROWS, HH), jnp.bfloat16),
            pltpu.VMEM((2, SROWS, HH), jnp.bfloat16),
            pltpu.VMEM((NQ - 1, SROWS, HH), jnp.bfloat16),
            pltpu.VMEM((NQ - 1, SROWS, HH), jnp.bfloat16),
            pltpu.VMEM((SROWS, H), jnp.float32),
            pltpu.VMEM((2, ROWS, H), jnp.bfloat16),
            pltpu.VMEM((NZ - 1, ROWS, H), jnp.bfloat16),
            pltpu.SemaphoreType.DMA((NQ - 1,)),
            pltpu.SemaphoreType.DMA((NQ - 1,)),
            pltpu.SemaphoreType.DMA((NQ - 1,)),
            pltpu.SemaphoreType.DMA((NQ - 1,)),
            pltpu.SemaphoreType.DMA((NZ - 1,)),
            pltpu.SemaphoreType.DMA((NZ - 1,)),
        ],
        compiler_params=pltpu.CompilerParams(collective_id=0),
    )(x4, gate4, expert_W, shared_own)
